# baseline (speedup 1.0000x reference)
import jax
import jax.numpy as jnp
from jax import lax
from jax.experimental import pallas as pl
from jax.experimental.pallas import tpu as pltpu
from jax.experimental.pallas import tpu_sc as plsc

NBIN = 256
L = 16
NW = 32
ROWS = 4096
RPW = ROWS // NW
RCH = 8
NCH = RPW // RCH
RV = 4096 // L
UNROLL = 16
HSTRIDE = 1041
HWORDS = L * HSTRIDE
C0_OFF = 0
G_OFF = 260
C1_OFF = 520


def _min_body(x_ref, o_ref):
    m = jnp.min(x_ref[...])

    @pl.when(pl.program_id(0) == 0)
    def _():
        o_ref[0, 0] = m

    @pl.when(pl.program_id(0) > 0)
    def _():
        o_ref[0, 0] = jnp.minimum(o_ref[0, 0], m)


def _tc_min(img0):
    return pl.pallas_call(
        _min_body,
        grid=(16,),
        in_specs=[pl.BlockSpec((256, 4096), lambda i: (i, 0))],
        out_specs=pl.BlockSpec(memory_space=pltpu.SMEM),
        out_shape=jax.ShapeDtypeStruct((1, 1), jnp.float32),
    )(img0)


def _sc_hist_body(img0_ref, img1_ref, min_ref, out_ref,
                  minbuf, hist, buf0, buf1, sem0, sem1):
    cid = lax.axis_index("c")
    sid = lax.axis_index("s")
    wid = sid * 2 + cid
    rowbase = wid * RPW

    zeros = jnp.zeros((L,), jnp.float32)
    ones = jnp.full((L,), 1.0, jnp.float32)

    @plsc.parallel_loop(0, HWORDS // L, unroll=8)
    def _zero(i):
        hist[pl.ds(i * L, L)] = zeros

    pltpu.sync_copy(min_ref, minbuf)
    hmin = minbuf[pl.ds(0, L)]
    inv_dh = (NBIN - 1.0) / (0.0 - hmin)
    lane_skew = lax.broadcasted_iota(jnp.int32, (L,), 0) * HSTRIDE

    bufs = (buf0, buf1)
    sems = (sem0, sem1)

    def _phase(img_ref, lanec):
        def _issue(c, b):
            @pl.when(c < NCH)
            def _():
                pltpu.async_copy(
                    img_ref.at[pl.ds(rowbase + c * RCH, RCH)],
                    bufs[b], sems[b])

        _issue(jnp.int32(0), 0)

        def _outer(c2, carry):
            for b in range(2):
                c = c2 * 2 + b
                _issue(c + 1, 1 - b)
                pltpu.make_async_copy(
                    img_ref.at[pl.ds(0, RCH)], bufs[b], sems[b]).wait()

                for r in range(RCH):
                    @plsc.parallel_loop(0, RV, unroll=UNROLL)
                    def _inner(j, _b=b, _r=r):
                        x = bufs[_b][_r, pl.ds(j * L, L)]
                        t = x * inv_dh + (NBIN - 1.0)
                        tcl = jnp.clip(t, 0.0, NBIN - 1.0)
                        keep = t == tcl
                        ti = tcl.astype(jnp.int32)
                        f = tcl - ti.astype(jnp.float32)
                        flc = lanec + ti
                        plsc.addupdate_scatter(hist, [flc], ones, mask=keep)
                        plsc.addupdate_scatter(
                            hist, [flc + G_OFF], f, mask=keep)
            return carry

        lax.fori_loop(0, NCH // 2, _outer, 0)

    _phase(img0_ref, lane_skew + C0_OFF)
    _phase(img1_ref, lane_skew + C1_OFF)

    pltpu.sync_copy(hist, out_ref.at[pl.ds(wid * HWORDS, HWORDS)])


def _sc_hist(img0, img1, minv):
    mesh = plsc.VectorSubcoreMesh(core_axis_name="c", subcore_axis_name="s")
    return pl.kernel(
        _sc_hist_body,
        out_type=jax.ShapeDtypeStruct((NW * HWORDS,), jnp.float32),
        mesh=mesh,
        scratch_types=[
            pltpu.VMEM((128,), jnp.float32),
            pltpu.VMEM((HWORDS,), jnp.float32),
            pltpu.VMEM((RCH, 4096), jnp.float32),
            pltpu.VMEM((RCH, 4096), jnp.float32),
            pltpu.SemaphoreType.DMA,
            pltpu.SemaphoreType.DMA,
        ],
        compiler_params=pltpu.CompilerParams(
            needs_layout_passes=False, use_tc_tiling_on_sc=True),
    )(img0, img1, minv)


def _kl_body(p_ref, o_ref):
    s = jnp.sum(p_ref[...], axis=0, keepdims=True)
    h0 = (s[:, C0_OFF:C0_OFF + NBIN]
          - s[:, C0_OFF + G_OFF:C0_OFF + G_OFF + NBIN]
          + s[:, C0_OFF + G_OFF - 1:C0_OFF + G_OFF - 1 + NBIN])
    h1 = (s[:, C1_OFF:C1_OFF + NBIN]
          - s[:, C1_OFF + G_OFF:C1_OFF + G_OFF + NBIN]
          + s[:, C1_OFF + G_OFF - 1:C1_OFF + G_OFF - 1 + NBIN])
    eps = 1e-10
    H0 = (h0 + eps) / (jnp.sum(h0) + eps)
    H1 = (h1 + eps) / (jnp.sum(h1) + eps)
    inp = jnp.log((H1 + eps) / H1)
    tgt = jnp.log((H1 + eps) / H0)
    o_ref[0, 0] = jnp.mean(jnp.exp(tgt) * (tgt - inp))


def _tc_kl(partials):
    return pl.pallas_call(
        _kl_body,
        in_specs=[pl.BlockSpec((NW * L, HSTRIDE), lambda: (0, 0))],
        out_specs=pl.BlockSpec(memory_space=pltpu.SMEM),
        out_shape=jax.ShapeDtypeStruct((1, 1), jnp.float32),
    )(partials)


@jax.jit
def kernel(img0, img1):
    min0 = _tc_min(img0)
    minv = jnp.broadcast_to(min0.reshape(()), (128,))
    partials = _sc_hist(img0, img1, minv)
    loss = _tc_kl(partials.reshape(NW * L, HSTRIDE))
    return loss[0, 0]


# speedup vs baseline: 1.0399x; 1.0399x over previous
import jax
import jax.numpy as jnp
from jax import lax
from jax.experimental import pallas as pl
from jax.experimental.pallas import tpu as pltpu
from jax.experimental.pallas import tpu_sc as plsc

NBIN = 256
L = 16
NW = 32
ROWS = 4096
RPW = ROWS // NW
RCH = 8
NCH = RPW // RCH
RV = 4096 // L
UNROLL = 8
HSTRIDE = 1041
HWORDS = L * HSTRIDE
C0_OFF = 0
G_OFF = 260
C1_OFF = 520


def _min_body(x_ref, o_ref):
    m = jnp.min(x_ref[...])

    @pl.when(pl.program_id(0) == 0)
    def _():
        o_ref[0, 0] = m

    @pl.when(pl.program_id(0) > 0)
    def _():
        o_ref[0, 0] = jnp.minimum(o_ref[0, 0], m)


def _tc_min(img0):
    return pl.pallas_call(
        _min_body,
        grid=(16,),
        in_specs=[pl.BlockSpec((256, 4096), lambda i: (i, 0))],
        out_specs=pl.BlockSpec(memory_space=pltpu.SMEM),
        out_shape=jax.ShapeDtypeStruct((1, 1), jnp.float32),
    )(img0)


def _sc_hist_body(img0_ref, img1_ref, min_ref, out_ref,
                  minbuf, hist, buf0, buf1, sem0, sem1):
    cid = lax.axis_index("c")
    sid = lax.axis_index("s")
    wid = sid * 2 + cid
    rowbase = wid * RPW

    zeros = jnp.zeros((L,), jnp.float32)
    ones = jnp.full((L,), 1.0, jnp.float32)

    @plsc.parallel_loop(0, HWORDS // L, unroll=8)
    def _zero(i):
        hist[pl.ds(i * L, L)] = zeros

    pltpu.sync_copy(min_ref, minbuf)
    hmin = minbuf[pl.ds(0, L)]
    inv_dh = (NBIN - 1.0) / (0.0 - hmin)
    lane_skew = lax.broadcasted_iota(jnp.int32, (L,), 0) * HSTRIDE

    bufs = (buf0, buf1)
    sems = (sem0, sem1)

    def _phase(img_ref, lanec):
        def _issue(c, b):
            @pl.when(c < NCH)
            def _():
                pltpu.async_copy(
                    img_ref.at[pl.ds(rowbase + c * RCH, RCH)],
                    bufs[b], sems[b])

        _issue(jnp.int32(0), 0)

        def _outer(c2, carry):
            for b in range(2):
                c = c2 * 2 + b
                _issue(c + 1, 1 - b)
                pltpu.make_async_copy(
                    img_ref.at[pl.ds(0, RCH)], bufs[b], sems[b]).wait()

                for r in range(RCH):
                    @plsc.parallel_loop(0, RV, unroll=UNROLL)
                    def _inner(j, _b=b, _r=r):
                        x = bufs[_b][_r, pl.ds(j * L, L)]
                        t = x * inv_dh + (NBIN - 1.0)
                        tcl = jnp.clip(t, 0.0, NBIN - 1.0)
                        keep = t == tcl
                        ti = tcl.astype(jnp.int32)
                        f = tcl - ti.astype(jnp.float32)
                        flc = lanec + ti
                        plsc.addupdate_scatter(hist, [flc], ones, mask=keep)
                        plsc.addupdate_scatter(
                            hist, [flc + G_OFF], f, mask=keep)
            return carry

        lax.fori_loop(0, NCH // 2, _outer, 0)

    _phase(img0_ref, lane_skew + C0_OFF)
    _phase(img1_ref, lane_skew + C1_OFF)

    pltpu.sync_copy(hist, out_ref.at[pl.ds(wid * HWORDS, HWORDS)])


def _sc_hist(img0, img1, minv):
    mesh = plsc.VectorSubcoreMesh(core_axis_name="c", subcore_axis_name="s")
    return pl.kernel(
        _sc_hist_body,
        out_type=jax.ShapeDtypeStruct((NW * HWORDS,), jnp.float32),
        mesh=mesh,
        scratch_types=[
            pltpu.VMEM((128,), jnp.float32),
            pltpu.VMEM((HWORDS,), jnp.float32),
            pltpu.VMEM((RCH, 4096), jnp.float32),
            pltpu.VMEM((RCH, 4096), jnp.float32),
            pltpu.SemaphoreType.DMA,
            pltpu.SemaphoreType.DMA,
        ],
        compiler_params=pltpu.CompilerParams(
            needs_layout_passes=False, use_tc_tiling_on_sc=True),
    )(img0, img1, minv)


def _kl_body(p_ref, o_ref):
    s = jnp.sum(p_ref[...], axis=0, keepdims=True)
    h0 = (s[:, C0_OFF:C0_OFF + NBIN]
          - s[:, C0_OFF + G_OFF:C0_OFF + G_OFF + NBIN]
          + s[:, C0_OFF + G_OFF - 1:C0_OFF + G_OFF - 1 + NBIN])
    h1 = (s[:, C1_OFF:C1_OFF + NBIN]
          - s[:, C1_OFF + G_OFF:C1_OFF + G_OFF + NBIN]
          + s[:, C1_OFF + G_OFF - 1:C1_OFF + G_OFF - 1 + NBIN])
    eps = 1e-10
    H0 = (h0 + eps) / (jnp.sum(h0) + eps)
    H1 = (h1 + eps) / (jnp.sum(h1) + eps)
    inp = jnp.log((H1 + eps) / H1)
    tgt = jnp.log((H1 + eps) / H0)
    o_ref[0, 0] = jnp.mean(jnp.exp(tgt) * (tgt - inp))


def _tc_kl(partials):
    return pl.pallas_call(
        _kl_body,
        in_specs=[pl.BlockSpec((NW * L, HSTRIDE), lambda: (0, 0))],
        out_specs=pl.BlockSpec(memory_space=pltpu.SMEM),
        out_shape=jax.ShapeDtypeStruct((1, 1), jnp.float32),
    )(partials)


@jax.jit
def kernel(img0, img1):
    min0 = _tc_min(img0)
    minv = jnp.broadcast_to(min0.reshape(()), (128,))
    partials = _sc_hist(img0, img1, minv)
    loss = _tc_kl(partials.reshape(NW * L, HSTRIDE))
    return loss[0, 0]


# speedup vs baseline: 1.1133x; 1.0705x over previous
import jax
import jax.numpy as jnp
from jax import lax
from jax.experimental import pallas as pl
from jax.experimental.pallas import tpu as pltpu
from jax.experimental.pallas import tpu_sc as plsc

NBIN = 256
L = 16
NW = 32
ROWS = 4096
RPW = ROWS // NW
RCH = 8
NCH = RPW // RCH
RV = 4096 // L
UNROLL = 8
HSTRIDE = 1049
HWORDS = L * HSTRIDE
C0_OFF = 0
G_OFF = 264
C1_OFF = 528


def _min_body(x_ref, o_ref):
    m = jnp.full((8, 128), jnp.min(x_ref[...]), jnp.float32)

    @pl.when(pl.program_id(0) == 0)
    def _():
        o_ref[...] = m

    @pl.when(pl.program_id(0) > 0)
    def _():
        o_ref[...] = jnp.minimum(o_ref[...], m)


def _tc_min(img0):
    return pl.pallas_call(
        _min_body,
        grid=(16,),
        in_specs=[pl.BlockSpec((256, 4096), lambda i: (i, 0))],
        out_specs=pl.BlockSpec((8, 128), lambda i: (0, 0)),
        out_shape=jax.ShapeDtypeStruct((8, 128), jnp.float32),
    )(img0)


def _sc_hist_body(img0_ref, img1_ref, min_ref, out_ref,
                  minbuf, hist, buf0, buf1, sem0, sem1):
    cid = lax.axis_index("c")
    sid = lax.axis_index("s")
    wid = sid * 2 + cid
    rowbase = wid * RPW

    zeros = jnp.zeros((L,), jnp.float32)
    ones = jnp.full((L,), 1.0, jnp.float32)

    @plsc.parallel_loop(0, HWORDS // L, unroll=8)
    def _zero(i):
        hist[pl.ds(i * L, L)] = zeros

    pltpu.sync_copy(min_ref, minbuf)
    hmin = minbuf[0, pl.ds(0, L)]
    inv_dh = (NBIN - 1.0) / (0.0 - hmin)
    lane_skew = lax.broadcasted_iota(jnp.int32, (L,), 0) * HSTRIDE

    bufs = (buf0, buf1)
    sems = (sem0, sem1)

    def _phase(img_ref, lanec):
        def _issue(c, b):
            @pl.when(c < NCH)
            def _():
                pltpu.async_copy(
                    img_ref.at[pl.ds(rowbase + c * RCH, RCH)],
                    bufs[b], sems[b])

        _issue(jnp.int32(0), 0)

        def _outer(c2, carry):
            for b in range(2):
                c = c2 * 2 + b
                _issue(c + 1, 1 - b)
                pltpu.make_async_copy(
                    img_ref.at[pl.ds(0, RCH)], bufs[b], sems[b]).wait()

                for r in range(RCH):
                    @plsc.parallel_loop(0, RV, unroll=UNROLL)
                    def _inner(j, _b=b, _r=r):
                        x = bufs[_b][_r, pl.ds(j * L, L)]
                        t = (x - hmin) * inv_dh
                        keep = plsc.bitcast(t, jnp.uint32) <= jnp.uint32(
                            0x437F0000)
                        ti = t.astype(jnp.int32)
                        f = t - ti.astype(jnp.float32)
                        flc = lanec + ti
                        plsc.addupdate_scatter(hist, [flc], ones, mask=keep)
                        plsc.addupdate_scatter(
                            hist.at[pl.ds(G_OFF, HWORDS - G_OFF)],
                            [flc], f, mask=keep)
            return carry

        lax.fori_loop(0, NCH // 2, _outer, 0)

    _phase(img0_ref, lane_skew + C0_OFF)
    _phase(img1_ref, lane_skew + C1_OFF)

    pltpu.sync_copy(hist, out_ref.at[pl.ds(wid * HWORDS, HWORDS)])


def _sc_hist(img0, img1, minv):
    mesh = plsc.VectorSubcoreMesh(core_axis_name="c", subcore_axis_name="s")
    return pl.kernel(
        _sc_hist_body,
        out_type=jax.ShapeDtypeStruct((NW * HWORDS,), jnp.float32),
        mesh=mesh,
        scratch_types=[
            pltpu.VMEM((8, 128), jnp.float32),
            pltpu.VMEM((HWORDS,), jnp.float32),
            pltpu.VMEM((RCH, 4096), jnp.float32),
            pltpu.VMEM((RCH, 4096), jnp.float32),
            pltpu.SemaphoreType.DMA,
            pltpu.SemaphoreType.DMA,
        ],
        compiler_params=pltpu.CompilerParams(
            needs_layout_passes=False, use_tc_tiling_on_sc=True),
    )(img0, img1, minv)


def _kl_body(p_ref, o_ref):
    s = jnp.sum(p_ref[...], axis=0, keepdims=True)
    h0 = (s[:, C0_OFF:C0_OFF + NBIN]
          - s[:, C0_OFF + G_OFF:C0_OFF + G_OFF + NBIN]
          + s[:, C0_OFF + G_OFF - 1:C0_OFF + G_OFF - 1 + NBIN])
    h1 = (s[:, C1_OFF:C1_OFF + NBIN]
          - s[:, C1_OFF + G_OFF:C1_OFF + G_OFF + NBIN]
          + s[:, C1_OFF + G_OFF - 1:C1_OFF + G_OFF - 1 + NBIN])
    eps = 1e-10
    H0 = (h0 + eps) / (jnp.sum(h0) + eps)
    H1 = (h1 + eps) / (jnp.sum(h1) + eps)
    inp = jnp.log((H1 + eps) / H1)
    tgt = jnp.log((H1 + eps) / H0)
    o_ref[0, 0] = jnp.mean(jnp.exp(tgt) * (tgt - inp))


def _tc_kl(partials):
    return pl.pallas_call(
        _kl_body,
        in_specs=[pl.BlockSpec((NW * L, HSTRIDE), lambda: (0, 0))],
        out_specs=pl.BlockSpec(memory_space=pltpu.SMEM),
        out_shape=jax.ShapeDtypeStruct((1, 1), jnp.float32),
    )(partials)


@jax.jit
def kernel(img0, img1):
    minv = _tc_min(img0)
    partials = _sc_hist(img0, img1, minv)
    loss = _tc_kl(partials.reshape(NW * L, HSTRIDE))
    return loss[0, 0]
